# baseline (speedup 1.0000x reference)
import functools

import jax
import jax.numpy as jnp
from jax import lax
from jax.experimental import pallas as pl
from jax.experimental.pallas import tpu as pltpu
from jax.experimental.pallas import tpu_sc as plsc

NN = 10000
D = 128
H = 256
ED = 16
EE = 320000
C = 128
CHT = 80
EP = 32 * CHT * C
CH0 = 56
CH1 = 2 * CHT - CH0
CHMX = max(CH0, CH1)
NP = 10240
NSUB = 16
RPT = NP // NSUB
EBLK = 2048
NBLK = 1000

_SQRT_HALF = 0.7071067811865476


def _gelu_exact(v):
    z = v * _SQRT_HALF
    a = jnp.abs(z)
    t = 1.0 / (1.0 + 0.3275911 * a)
    poly = t * (0.254829592 + t * (-0.284496736 + t * (
        1.421413741 + t * (-1.453152027 + t * 1.061405429))))
    erf_z = jnp.sign(z) * (1.0 - poly * jnp.exp(-a * a))
    return v * 0.5 * (1.0 + erf_z)


def _rtne_bf16_bits(x):
    u = lax.bitcast_convert_type(x, jnp.int32)
    return u + 0x7FFF + (lax.shift_right_logical(u, 16) & 1)

def _pack_bf16_pair(lo, hi):
    lo_b = lax.shift_right_logical(_rtne_bf16_bits(lo), 16)
    hi_b = _rtne_bf16_bits(hi) & jnp.int32(-65536)
    return lo_b | hi_b


def _unpack_bf16_pair(w):
    lo = lax.bitcast_convert_type(lax.shift_left(w, 16), jnp.float32)
    hi = lax.bitcast_convert_type(w & jnp.int32(-65536), jnp.float32)
    return lo, hi


def _k1_body(x_ref, wa_ref, wb_ref, p_ref, q_ref):
    xb = x_ref[...]
    p = jnp.dot(xb, wa_ref[...], preferred_element_type=jnp.float32)
    q = jnp.dot(xb, wb_ref[...], preferred_element_type=jnp.float32)
    p_ref[...] = _pack_bf16_pair(p[:, :D], p[:, D:])
    q_ref[...] = _pack_bf16_pair(q[:, :D], q[:, D:])


_k1 = pl.pallas_call(
    _k1_body,
    grid=(NP // 1024,),
    in_specs=[
        pl.BlockSpec((1024, D), lambda i: (i, 0)),
        pl.BlockSpec((D, H), lambda i: (0, 0)),
        pl.BlockSpec((D, H), lambda i: (0, 0)),
    ],
    out_specs=[
        pl.BlockSpec((1024, H // 2), lambda i: (i, 0)),
        pl.BlockSpec((1024, H // 2), lambda i: (i, 0)),
    ],
    out_shape=[jax.ShapeDtypeStruct((NP, H // 2), jnp.int32)] * 2,
)


def _s1_body(p_hbm, q_hbm, row_hbm, col_hbm, tp_hbm, tq_hbm,
             row_v, col_v, pg, qg, sem_p0, sem_q0, sem_p1, sem_q1, sem_o):
    cid = lax.axis_index("c")
    sid = lax.axis_index("s")
    cbase = jnp.where(cid == 0, sid * CH0, NSUB * CH0 + sid * CH1)
    nch = jnp.where(cid == 0, CH0, CH1)

    @pl.when(cid == 0)
    def _():
        pltpu.sync_copy(row_hbm.at[pl.ds(cbase, CH0)], row_v.at[pl.ds(0, CH0)])
        pltpu.sync_copy(col_hbm.at[pl.ds(cbase, CH0)], col_v.at[pl.ds(0, CH0)])

    @pl.when(cid == 1)
    def _():
        pltpu.sync_copy(row_hbm.at[pl.ds(cbase, CH1)], row_v.at[pl.ds(0, CH1)])
        pltpu.sync_copy(col_hbm.at[pl.ds(cbase, CH1)], col_v.at[pl.ds(0, CH1)])

    sems = ((sem_p0, sem_q0), (sem_p1, sem_q1))

    pltpu.async_copy(p_hbm.at[row_v.at[0]], pg.at[0], sem_p0)
    pltpu.async_copy(q_hbm.at[col_v.at[0]], qg.at[0], sem_q0)

    def chunk2(jj, carry):
        for b in (0, 1):
            j = jj * 2 + b
            nb = 1 - b

            @pl.when(j + 1 < nch)
            def _():
                pltpu.async_copy(
                    p_hbm.at[row_v.at[j + 1]], pg.at[nb], sems[nb][0])
                pltpu.async_copy(
                    q_hbm.at[col_v.at[j + 1]], qg.at[nb], sems[nb][1])

            pltpu.make_async_copy(
                p_hbm.at[row_v.at[j]], pg.at[b], sems[b][0]).wait()
            pltpu.make_async_copy(
                q_hbm.at[col_v.at[j]], qg.at[b], sems[b][1]).wait()
            co1 = pltpu.async_copy(
                pg.at[b], tp_hbm.at[pl.ds((cbase + j) * C, C)], sem_o)
            co2 = pltpu.async_copy(
                qg.at[b], tq_hbm.at[pl.ds((cbase + j) * C, C)], sem_o)
            co1.wait()
            co2.wait()
        return carry

    lax.fori_loop(0, nch // 2, chunk2, 0)


_s1 = pl.kernel(
    _s1_body,
    out_type=(
        jax.ShapeDtypeStruct((EP, H // 2), jnp.int32),
        jax.ShapeDtypeStruct((EP, H // 2), jnp.int32),
    ),
    mesh=plsc.VectorSubcoreMesh(core_axis_name="c", subcore_axis_name="s"),
    scratch_types=[
        pltpu.VMEM((CHMX, C), jnp.int32),
        pltpu.VMEM((CHMX, C), jnp.int32),
        pltpu.VMEM((2, C, H // 2), jnp.int32),
        pltpu.VMEM((2, C, H // 2), jnp.int32),
        pltpu.SemaphoreType.DMA,
        pltpu.SemaphoreType.DMA,
        pltpu.SemaphoreType.DMA,
        pltpu.SemaphoreType.DMA,
        pltpu.SemaphoreType.DMA,
    ],
)


def _gelu_tanh(v):
    return 0.5 * v * (1.0 + jnp.tanh(0.7978845608028654
                                     * (v + 0.044715 * v * v * v)))


def _k2_body(tp_ref, tq_ref, ea_ref, wc_ref, b1_ref, u0_ref, u1_ref):
    p0, p1 = _unpack_bf16_pair(tp_ref[...])
    q0, q1 = _unpack_bf16_pair(tq_ref[...])
    r = jnp.dot(ea_ref[...], wc_ref[...], preferred_element_type=jnp.float32)
    b1 = b1_ref[...]
    u0_ref[...] = _gelu_tanh(p0 + q0 + r[:, :D] + b1[:, :D])
    u1_ref[...] = _gelu_tanh(p1 + q1 + r[:, D:] + b1[:, D:])


_k2 = pl.pallas_call(
    _k2_body,
    grid=(EP // EBLK,),
    in_specs=[
        pl.BlockSpec((EBLK, H // 2), lambda i: (i, 0)),
        pl.BlockSpec((EBLK, H // 2), lambda i: (i, 0)),
        pl.BlockSpec((EBLK, ED), lambda i: (i, 0)),
        pl.BlockSpec((ED, H), lambda i: (0, 0)),
        pl.BlockSpec((1, H), lambda i: (0, 0)),
    ],
    out_specs=[
        pl.BlockSpec((EBLK, D), lambda i: (i, 0)),
        pl.BlockSpec((EBLK, D), lambda i: (i, 0)),
    ],
    out_shape=[jax.ShapeDtypeStruct((EP, D), jnp.float32)] * 2,
)


def _s2_body(u0_hbm, u1_hbm, col_hbm, part0_hbm, part1_hbm, pcnt_hbm,
             col_v, ub, acc, sem0, sem1):
    cid = lax.axis_index("c")
    sid = lax.axis_index("s")
    wid = cid * NSUB + sid
    pltpu.sync_copy(col_hbm.at[pl.ds(wid * CHT, CHT)], col_v)

    zero16 = jnp.zeros((16,), jnp.float32)
    ones16 = jnp.ones((16,), jnp.float32)
    sems = (sem0, sem1)

    def _fill(b, val):
        def frow(r, c2):
            for k in range(D // 16):
                ub[b, r, pl.ds(k * 16, 16)] = val
            return c2

        lax.fori_loop(0, C, frow, 0)

    for h in range(3):
        _fill(0, zero16)
        for k in range(RPT // C):
            pltpu.sync_copy(ub.at[0], acc.at[pl.ds(sid * RPT + k * C, C)])
        if h == 2:
            _fill(0, ones16)
        plsc.subcore_barrier()
        u_hbm = (u0_hbm, u1_hbm, None)[h]

        if u_hbm is None:
            def chunk(j, carry):
                pltpu.sync_copy(ub.at[0], acc.at[col_v.at[j]], add=True)
                return carry

            lax.fori_loop(0, CHT, chunk, 0)
        else:
            pltpu.async_copy(
                u_hbm.at[pl.ds(wid * CHT * C, C)], ub.at[0], sem0)

            def chunk2(jj, carry):
                for b in (0, 1):
                    j = jj * 2 + b

                    @pl.when(j + 1 < CHT)
                    def _():
                        pltpu.async_copy(
                            u_hbm.at[pl.ds((wid * CHT + j + 1) * C, C)],
                            ub.at[1 - b], sems[1 - b])

                    pltpu.make_async_copy(
                        u_hbm.at[pl.ds((wid * CHT + j) * C, C)],
                        ub.at[b], sems[b]).wait()
                    pltpu.sync_copy(ub.at[b], acc.at[col_v.at[j]], add=True)
                return carry

            lax.fori_loop(0, CHT // 2, chunk2, 0)
        plsc.subcore_barrier()
        part = (part0_hbm, part1_hbm, pcnt_hbm)[h]
        pltpu.sync_copy(acc.at[pl.ds(sid * RPT, RPT)],
                        part.at[cid, pl.ds(sid * RPT, RPT)])


_s2 = pl.kernel(
    _s2_body,
    out_type=(
        jax.ShapeDtypeStruct((2, NP, D), jnp.float32),
        jax.ShapeDtypeStruct((2, NP, D), jnp.float32),
        jax.ShapeDtypeStruct((2, NP, D), jnp.float32),
    ),
    mesh=plsc.VectorSubcoreMesh(core_axis_name="c", subcore_axis_name="s"),
    scratch_types=[
        pltpu.VMEM((CHT, C), jnp.int32),
        pltpu.VMEM((2, C, D), jnp.float32),
        pltpu.VMEM_SHARED((NP, D), jnp.float32),
        pltpu.SemaphoreType.DMA,
        pltpu.SemaphoreType.DMA,
    ],
)


def _k3_body(x_ref, p0_ref, p1_ref, pc_ref, w2_ref, b2_ref,
             ua_ref, uba_ref, ub2w_ref, ubb_ref, g_ref, be_ref, o_ref):
    s0 = p0_ref[0] + p0_ref[1]
    s1 = p1_ref[0] + p1_ref[1]
    cnt = pc_ref[0, :, 0:1] + pc_ref[1, :, 0:1]
    s = jnp.concatenate([s0, s1], axis=1)
    aggr = (jnp.dot(s, w2_ref[...], preferred_element_type=jnp.float32)
            + cnt * b2_ref[...]) / jnp.maximum(cnt, 1.0)
    xb = x_ref[...]
    u_in = jnp.concatenate([xb, aggr], axis=1)
    h2 = _gelu_exact(jnp.dot(u_in, ua_ref[...],
                             preferred_element_type=jnp.float32) + uba_ref[...])
    out = jnp.dot(h2, ub2w_ref[...],
                  preferred_element_type=jnp.float32) + ubb_ref[...]
    z = xb + out
    mu = jnp.mean(z, axis=1, keepdims=True)
    dz = z - mu
    var = jnp.mean(dz * dz, axis=1, keepdims=True)
    o_ref[...] = dz * lax.rsqrt(var + 1e-5) * g_ref[...] + be_ref[...]


_k3 = pl.pallas_call(
    _k3_body,
    grid=(NN // NBLK,),
    in_specs=[
        pl.BlockSpec((NBLK, D), lambda i: (i, 0)),
        pl.BlockSpec((2, NBLK, D), lambda i: (0, i, 0)),
        pl.BlockSpec((2, NBLK, D), lambda i: (0, i, 0)),
        pl.BlockSpec((2, NBLK, D), lambda i: (0, i, 0)),
        pl.BlockSpec((H, D), lambda i: (0, 0)),
        pl.BlockSpec((1, D), lambda i: (0, 0)),
        pl.BlockSpec((H, D), lambda i: (0, 0)),
        pl.BlockSpec((1, D), lambda i: (0, 0)),
        pl.BlockSpec((D, D), lambda i: (0, 0)),
        pl.BlockSpec((1, D), lambda i: (0, 0)),
        pl.BlockSpec((1, D), lambda i: (0, 0)),
        pl.BlockSpec((1, D), lambda i: (0, 0)),
    ],
    out_specs=pl.BlockSpec((NBLK, D), lambda i: (i, 0)),
    out_shape=jax.ShapeDtypeStruct((NN, D), jnp.float32),
)


def kernel(x, edge_index, edge_attr, W1, b1, W2, b2, U1, ub1, U2, ub2,
           gamma, beta):
    f32 = jnp.float32
    row = edge_index[0]
    col = edge_index[1]
    rowp = jnp.concatenate(
        [row, jnp.zeros((EP - EE,), jnp.int32)]).reshape(EP // C, C)
    colp = jnp.concatenate(
        [col, jnp.full((EP - EE,), NN, jnp.int32)]).reshape(EP // C, C)
    eap = jnp.concatenate([edge_attr, jnp.zeros((EP - EE, ED), f32)], axis=0)
    xp = jnp.zeros((NP, D), f32).at[:NN].set(x)
    W1a = W1[:D]
    W1b = W1[D:2 * D]
    W1c = W1[2 * D:]
    P, Q = _k1(xp, W1a, W1b)
    Tp, Tq = _s1(P, Q, rowp, colp)
    Uh0, Uh1 = _k2(Tp, Tq, eap, W1c, b1.reshape(1, H))
    p0, p1, pc = _s2(Uh0, Uh1, colp)
    return _k3(x, p0, p1, pc, W2, b2.reshape(1, D), U1, ub1.reshape(1, D),
               U2, ub2.reshape(1, D), gamma.reshape(1, D), beta.reshape(1, D))


# speedup vs baseline: 1.0085x; 1.0085x over previous
import functools

import jax
import jax.numpy as jnp
from jax import lax
from jax.experimental import pallas as pl
from jax.experimental.pallas import tpu as pltpu
from jax.experimental.pallas import tpu_sc as plsc

NN = 10000
D = 128
H = 256
ED = 16
EE = 320000
C = 128
CHT = 80
EP = 32 * CHT * C
CH0 = 160
CH1 = 2 * CHT - CH0
CHMX = max(CH0, CH1)
NP = 10240
NSUB = 16
RPT = NP // NSUB
EBLK = 2048
NBLK = 1000

_SQRT_HALF = 0.7071067811865476


def _gelu_exact(v):
    z = v * _SQRT_HALF
    a = jnp.abs(z)
    t = 1.0 / (1.0 + 0.3275911 * a)
    poly = t * (0.254829592 + t * (-0.284496736 + t * (
        1.421413741 + t * (-1.453152027 + t * 1.061405429))))
    erf_z = jnp.sign(z) * (1.0 - poly * jnp.exp(-a * a))
    return v * 0.5 * (1.0 + erf_z)


def _rtne_bf16_bits(x):
    u = lax.bitcast_convert_type(x, jnp.int32)
    return u + 0x7FFF + (lax.shift_right_logical(u, 16) & 1)

def _pack_bf16_pair(lo, hi):
    lo_b = lax.shift_right_logical(_rtne_bf16_bits(lo), 16)
    hi_b = _rtne_bf16_bits(hi) & jnp.int32(-65536)
    return lo_b | hi_b


def _unpack_bf16_pair(w):
    lo = lax.bitcast_convert_type(lax.shift_left(w, 16), jnp.float32)
    hi = lax.bitcast_convert_type(w & jnp.int32(-65536), jnp.float32)
    return lo, hi


def _k1_body(x_ref, wa_ref, wb_ref, p_ref, q_ref):
    xb = x_ref[...]
    p = jnp.dot(xb, wa_ref[...], preferred_element_type=jnp.float32)
    q = jnp.dot(xb, wb_ref[...], preferred_element_type=jnp.float32)
    p_ref[...] = _pack_bf16_pair(p[:, :D], p[:, D:])
    q_ref[...] = _pack_bf16_pair(q[:, :D], q[:, D:])


_k1 = pl.pallas_call(
    _k1_body,
    grid=(NP // 1024,),
    in_specs=[
        pl.BlockSpec((1024, D), lambda i: (i, 0)),
        pl.BlockSpec((D, H), lambda i: (0, 0)),
        pl.BlockSpec((D, H), lambda i: (0, 0)),
    ],
    out_specs=[
        pl.BlockSpec((1024, H // 2), lambda i: (i, 0)),
        pl.BlockSpec((1024, H // 2), lambda i: (i, 0)),
    ],
    out_shape=[jax.ShapeDtypeStruct((NP, H // 2), jnp.int32)] * 2,
)


def _s1_body(p_hbm, q_hbm, row_hbm, col_hbm, tp_hbm, tq_hbm,
             row_v, col_v, pg, qg, sem_p0, sem_q0, sem_p1, sem_q1, sem_o):
    cid = lax.axis_index("c")
    sid = lax.axis_index("s")
    cbase = jnp.where(cid == 0, sid * CH0, NSUB * CH0 + sid * CH1)
    nch = jnp.where(cid == 0, CH0, CH1)

    @pl.when(cid == 0)
    def _():
        pltpu.sync_copy(row_hbm.at[pl.ds(cbase, CH0)], row_v.at[pl.ds(0, CH0)])
        pltpu.sync_copy(col_hbm.at[pl.ds(cbase, CH0)], col_v.at[pl.ds(0, CH0)])

    if CH1 > 0:
        @pl.when(cid == 1)
        def _():
            pltpu.sync_copy(row_hbm.at[pl.ds(cbase, CH1)],
                            row_v.at[pl.ds(0, CH1)])
            pltpu.sync_copy(col_hbm.at[pl.ds(cbase, CH1)],
                            col_v.at[pl.ds(0, CH1)])

    sems = ((sem_p0, sem_q0), (sem_p1, sem_q1))

    @pl.when(nch > 0)
    def _():
        pltpu.async_copy(p_hbm.at[row_v.at[0]], pg.at[0], sem_p0)
        pltpu.async_copy(q_hbm.at[col_v.at[0]], qg.at[0], sem_q0)

    def chunk2(jj, carry):
        for b in (0, 1):
            j = jj * 2 + b
            nb = 1 - b

            @pl.when(j + 1 < nch)
            def _():
                pltpu.async_copy(
                    p_hbm.at[row_v.at[j + 1]], pg.at[nb], sems[nb][0])
                pltpu.async_copy(
                    q_hbm.at[col_v.at[j + 1]], qg.at[nb], sems[nb][1])

            pltpu.make_async_copy(
                p_hbm.at[row_v.at[j]], pg.at[b], sems[b][0]).wait()
            pltpu.make_async_copy(
                q_hbm.at[col_v.at[j]], qg.at[b], sems[b][1]).wait()
            co1 = pltpu.async_copy(
                pg.at[b], tp_hbm.at[pl.ds((cbase + j) * C, C)], sem_o)
            co2 = pltpu.async_copy(
                qg.at[b], tq_hbm.at[pl.ds((cbase + j) * C, C)], sem_o)
            co1.wait()
            co2.wait()
        return carry

    lax.fori_loop(0, nch // 2, chunk2, 0)


_s1 = pl.kernel(
    _s1_body,
    out_type=(
        jax.ShapeDtypeStruct((EP, H // 2), jnp.int32),
        jax.ShapeDtypeStruct((EP, H // 2), jnp.int32),
    ),
    mesh=plsc.VectorSubcoreMesh(core_axis_name="c", subcore_axis_name="s"),
    scratch_types=[
        pltpu.VMEM((CHMX, C), jnp.int32),
        pltpu.VMEM((CHMX, C), jnp.int32),
        pltpu.VMEM((2, C, H // 2), jnp.int32),
        pltpu.VMEM((2, C, H // 2), jnp.int32),
        pltpu.SemaphoreType.DMA,
        pltpu.SemaphoreType.DMA,
        pltpu.SemaphoreType.DMA,
        pltpu.SemaphoreType.DMA,
        pltpu.SemaphoreType.DMA,
    ],
)


def _gelu_tanh(v):
    return 0.5 * v * (1.0 + jnp.tanh(0.7978845608028654
                                     * (v + 0.044715 * v * v * v)))


def _k2_body(tp_ref, tq_ref, ea_ref, wc_ref, b1_ref, u0_ref, u1_ref):
    p0, p1 = _unpack_bf16_pair(tp_ref[...])
    q0, q1 = _unpack_bf16_pair(tq_ref[...])
    r = jnp.dot(ea_ref[...], wc_ref[...], preferred_element_type=jnp.float32)
    b1 = b1_ref[...]
    u0_ref[...] = _gelu_tanh(p0 + q0 + r[:, :D] + b1[:, :D])
    u1_ref[...] = _gelu_tanh(p1 + q1 + r[:, D:] + b1[:, D:])


_k2 = pl.pallas_call(
    _k2_body,
    grid=(EP // EBLK,),
    in_specs=[
        pl.BlockSpec((EBLK, H // 2), lambda i: (i, 0)),
        pl.BlockSpec((EBLK, H // 2), lambda i: (i, 0)),
        pl.BlockSpec((EBLK, ED), lambda i: (i, 0)),
        pl.BlockSpec((ED, H), lambda i: (0, 0)),
        pl.BlockSpec((1, H), lambda i: (0, 0)),
    ],
    out_specs=[
        pl.BlockSpec((EBLK, D), lambda i: (i, 0)),
        pl.BlockSpec((EBLK, D), lambda i: (i, 0)),
    ],
    out_shape=[jax.ShapeDtypeStruct((EP, D), jnp.float32)] * 2,
)


def _s2_body(u0_hbm, u1_hbm, col_hbm, part0_hbm, part1_hbm, pcnt_hbm,
             col_v, ub, acc, sem0, sem1):
    cid = lax.axis_index("c")
    sid = lax.axis_index("s")
    wid = cid * NSUB + sid
    pltpu.sync_copy(col_hbm.at[pl.ds(wid * CHT, CHT)], col_v)

    zero16 = jnp.zeros((16,), jnp.float32)
    ones16 = jnp.ones((16,), jnp.float32)
    sems = (sem0, sem1)

    def _fill(b, val):
        def frow(r, c2):
            for k in range(D // 16):
                ub[b, r, pl.ds(k * 16, 16)] = val
            return c2

        lax.fori_loop(0, C, frow, 0)

    for h in range(3):
        _fill(0, zero16)
        for k in range(RPT // C):
            pltpu.sync_copy(ub.at[0], acc.at[pl.ds(sid * RPT + k * C, C)])
        if h == 2:
            _fill(0, ones16)
        plsc.subcore_barrier()
        u_hbm = (u0_hbm, u1_hbm, None)[h]

        if u_hbm is None:
            def chunk(j, carry):
                pltpu.sync_copy(ub.at[0], acc.at[col_v.at[j]], add=True)
                return carry

            lax.fori_loop(0, CHT, chunk, 0)
        else:
            pltpu.async_copy(
                u_hbm.at[pl.ds(wid * CHT * C, C)], ub.at[0], sem0)

            def chunk2(jj, carry):
                for b in (0, 1):
                    j = jj * 2 + b

                    @pl.when(j + 1 < CHT)
                    def _():
                        pltpu.async_copy(
                            u_hbm.at[pl.ds((wid * CHT + j + 1) * C, C)],
                            ub.at[1 - b], sems[1 - b])

                    pltpu.make_async_copy(
                        u_hbm.at[pl.ds((wid * CHT + j) * C, C)],
                        ub.at[b], sems[b]).wait()
                    pltpu.sync_copy(ub.at[b], acc.at[col_v.at[j]], add=True)
                return carry

            lax.fori_loop(0, CHT // 2, chunk2, 0)
        plsc.subcore_barrier()
        part = (part0_hbm, part1_hbm, pcnt_hbm)[h]
        pltpu.sync_copy(acc.at[pl.ds(sid * RPT, RPT)],
                        part.at[cid, pl.ds(sid * RPT, RPT)])


_s2 = pl.kernel(
    _s2_body,
    out_type=(
        jax.ShapeDtypeStruct((2, NP, D), jnp.float32),
        jax.ShapeDtypeStruct((2, NP, D), jnp.float32),
        jax.ShapeDtypeStruct((2, NP, D), jnp.float32),
    ),
    mesh=plsc.VectorSubcoreMesh(core_axis_name="c", subcore_axis_name="s"),
    scratch_types=[
        pltpu.VMEM((CHT, C), jnp.int32),
        pltpu.VMEM((2, C, D), jnp.float32),
        pltpu.VMEM_SHARED((NP, D), jnp.float32),
        pltpu.SemaphoreType.DMA,
        pltpu.SemaphoreType.DMA,
    ],
)


def _k3_body(x_ref, p0_ref, p1_ref, pc_ref, w2_ref, b2_ref,
             ua_ref, uba_ref, ub2w_ref, ubb_ref, g_ref, be_ref, o_ref):
    s0 = p0_ref[0] + p0_ref[1]
    s1 = p1_ref[0] + p1_ref[1]
    cnt = pc_ref[0, :, 0:1] + pc_ref[1, :, 0:1]
    s = jnp.concatenate([s0, s1], axis=1)
    aggr = (jnp.dot(s, w2_ref[...], preferred_element_type=jnp.float32)
            + cnt * b2_ref[...]) / jnp.maximum(cnt, 1.0)
    xb = x_ref[...]
    u_in = jnp.concatenate([xb, aggr], axis=1)
    h2 = _gelu_exact(jnp.dot(u_in, ua_ref[...],
                             preferred_element_type=jnp.float32) + uba_ref[...])
    out = jnp.dot(h2, ub2w_ref[...],
                  preferred_element_type=jnp.float32) + ubb_ref[...]
    z = xb + out
    mu = jnp.mean(z, axis=1, keepdims=True)
    dz = z - mu
    var = jnp.mean(dz * dz, axis=1, keepdims=True)
    o_ref[...] = dz * lax.rsqrt(var + 1e-5) * g_ref[...] + be_ref[...]


_k3 = pl.pallas_call(
    _k3_body,
    grid=(NN // NBLK,),
    in_specs=[
        pl.BlockSpec((NBLK, D), lambda i: (i, 0)),
        pl.BlockSpec((2, NBLK, D), lambda i: (0, i, 0)),
        pl.BlockSpec((2, NBLK, D), lambda i: (0, i, 0)),
        pl.BlockSpec((2, NBLK, D), lambda i: (0, i, 0)),
        pl.BlockSpec((H, D), lambda i: (0, 0)),
        pl.BlockSpec((1, D), lambda i: (0, 0)),
        pl.BlockSpec((H, D), lambda i: (0, 0)),
        pl.BlockSpec((1, D), lambda i: (0, 0)),
        pl.BlockSpec((D, D), lambda i: (0, 0)),
        pl.BlockSpec((1, D), lambda i: (0, 0)),
        pl.BlockSpec((1, D), lambda i: (0, 0)),
        pl.BlockSpec((1, D), lambda i: (0, 0)),
    ],
    out_specs=pl.BlockSpec((NBLK, D), lambda i: (i, 0)),
    out_shape=jax.ShapeDtypeStruct((NN, D), jnp.float32),
)


def kernel(x, edge_index, edge_attr, W1, b1, W2, b2, U1, ub1, U2, ub2,
           gamma, beta):
    f32 = jnp.float32
    row = edge_index[0]
    col = edge_index[1]
    rowp = jnp.concatenate(
        [row, jnp.zeros((EP - EE,), jnp.int32)]).reshape(EP // C, C)
    colp = jnp.concatenate(
        [col, jnp.full((EP - EE,), NN, jnp.int32)]).reshape(EP // C, C)
    eap = jnp.concatenate([edge_attr, jnp.zeros((EP - EE, ED), f32)], axis=0)
    xp = jnp.zeros((NP, D), f32).at[:NN].set(x)
    W1a = W1[:D]
    W1b = W1[D:2 * D]
    W1c = W1[2 * D:]
    P, Q = _k1(xp, W1a, W1b)
    Tp, Tq = _s1(P, Q, rowp, colp)
    Uh0, Uh1 = _k2(Tp, Tq, eap, W1c, b1.reshape(1, H))
    p0, p1, pc = _s2(Uh0, Uh1, colp)
    return _k3(x, p0, p1, pc, W2, b2.reshape(1, D), U1, ub1.reshape(1, D),
               U2, ub2.reshape(1, D), gamma.reshape(1, D), beta.reshape(1, D))


# speedup vs baseline: 1.0232x; 1.0146x over previous
import functools

import jax
import jax.numpy as jnp
from jax import lax
from jax.experimental import pallas as pl
from jax.experimental.pallas import tpu as pltpu
from jax.experimental.pallas import tpu_sc as plsc

NN = 10000
D = 128
H = 256
ED = 16
EE = 320000
C = 128
CHT = 80
EP = 32 * CHT * C
CH0 = 104
CH1 = 2 * CHT - CH0
CHMX = max(CH0, CH1)
NP = 10240
NSUB = 16
RPT = NP // NSUB
EBLK = 2048
NBLK = 1000

_SQRT_HALF = 0.7071067811865476


def _gelu_exact(v):
    z = v * _SQRT_HALF
    a = jnp.abs(z)
    t = 1.0 / (1.0 + 0.3275911 * a)
    poly = t * (0.254829592 + t * (-0.284496736 + t * (
        1.421413741 + t * (-1.453152027 + t * 1.061405429))))
    erf_z = jnp.sign(z) * (1.0 - poly * jnp.exp(-a * a))
    return v * 0.5 * (1.0 + erf_z)


def _rtne_bf16_bits(x):
    u = lax.bitcast_convert_type(x, jnp.int32)
    return u + 0x7FFF + (lax.shift_right_logical(u, 16) & 1)

def _pack_bf16_pair(lo, hi):
    lo_b = lax.shift_right_logical(_rtne_bf16_bits(lo), 16)
    hi_b = _rtne_bf16_bits(hi) & jnp.int32(-65536)
    return lo_b | hi_b


def _unpack_bf16_pair(w):
    lo = lax.bitcast_convert_type(lax.shift_left(w, 16), jnp.float32)
    hi = lax.bitcast_convert_type(w & jnp.int32(-65536), jnp.float32)
    return lo, hi


def _k1_body(x_ref, wa_ref, wb_ref, p_ref, q_ref):
    xb = x_ref[...]
    p = jnp.dot(xb, wa_ref[...], preferred_element_type=jnp.float32)
    q = jnp.dot(xb, wb_ref[...], preferred_element_type=jnp.float32)
    p_ref[...] = _pack_bf16_pair(p[:, :D], p[:, D:])
    q_ref[...] = _pack_bf16_pair(q[:, :D], q[:, D:])


_k1 = pl.pallas_call(
    _k1_body,
    grid=(NP // 1024,),
    in_specs=[
        pl.BlockSpec((1024, D), lambda i: (i, 0)),
        pl.BlockSpec((D, H), lambda i: (0, 0)),
        pl.BlockSpec((D, H), lambda i: (0, 0)),
    ],
    out_specs=[
        pl.BlockSpec((1024, H // 2), lambda i: (i, 0)),
        pl.BlockSpec((1024, H // 2), lambda i: (i, 0)),
    ],
    out_shape=[jax.ShapeDtypeStruct((NP, H // 2), jnp.int32)] * 2,
)


def _s1_body(p_hbm, q_hbm, row_hbm, col_hbm, tp_hbm, tq_hbm,
             row_v, col_v, pg, qg, sem_p0, sem_q0, sem_p1, sem_q1, sem_o):
    cid = lax.axis_index("c")
    sid = lax.axis_index("s")
    cbase = jnp.where(cid == 0, sid * CH0, NSUB * CH0 + sid * CH1)
    nch = jnp.where(cid == 0, CH0, CH1)

    @pl.when(cid == 0)
    def _():
        pltpu.sync_copy(row_hbm.at[pl.ds(cbase, CH0)], row_v.at[pl.ds(0, CH0)])
        pltpu.sync_copy(col_hbm.at[pl.ds(cbase, CH0)], col_v.at[pl.ds(0, CH0)])

    if CH1 > 0:
        @pl.when(cid == 1)
        def _():
            pltpu.sync_copy(row_hbm.at[pl.ds(cbase, CH1)],
                            row_v.at[pl.ds(0, CH1)])
            pltpu.sync_copy(col_hbm.at[pl.ds(cbase, CH1)],
                            col_v.at[pl.ds(0, CH1)])

    sems = ((sem_p0, sem_q0), (sem_p1, sem_q1))

    @pl.when(nch > 0)
    def _():
        pltpu.async_copy(p_hbm.at[row_v.at[0]], pg.at[0], sem_p0)
        pltpu.async_copy(q_hbm.at[col_v.at[0]], qg.at[0], sem_q0)

    def chunk2(jj, carry):
        for b in (0, 1):
            j = jj * 2 + b
            nb = 1 - b

            @pl.when(j + 1 < nch)
            def _():
                pltpu.async_copy(
                    p_hbm.at[row_v.at[j + 1]], pg.at[nb], sems[nb][0])
                pltpu.async_copy(
                    q_hbm.at[col_v.at[j + 1]], qg.at[nb], sems[nb][1])

            pltpu.make_async_copy(
                p_hbm.at[row_v.at[j]], pg.at[b], sems[b][0]).wait()
            pltpu.make_async_copy(
                q_hbm.at[col_v.at[j]], qg.at[b], sems[b][1]).wait()
            co1 = pltpu.async_copy(
                pg.at[b], tp_hbm.at[pl.ds((cbase + j) * C, C)], sem_o)
            co2 = pltpu.async_copy(
                qg.at[b], tq_hbm.at[pl.ds((cbase + j) * C, C)], sem_o)
            co1.wait()
            co2.wait()
        return carry

    lax.fori_loop(0, nch // 2, chunk2, 0)


_s1 = pl.kernel(
    _s1_body,
    out_type=(
        jax.ShapeDtypeStruct((EP, H // 2), jnp.int32),
        jax.ShapeDtypeStruct((EP, H // 2), jnp.int32),
    ),
    mesh=plsc.VectorSubcoreMesh(core_axis_name="c", subcore_axis_name="s"),
    scratch_types=[
        pltpu.VMEM((CHMX, C), jnp.int32),
        pltpu.VMEM((CHMX, C), jnp.int32),
        pltpu.VMEM((2, C, H // 2), jnp.int32),
        pltpu.VMEM((2, C, H // 2), jnp.int32),
        pltpu.SemaphoreType.DMA,
        pltpu.SemaphoreType.DMA,
        pltpu.SemaphoreType.DMA,
        pltpu.SemaphoreType.DMA,
        pltpu.SemaphoreType.DMA,
    ],
)


def _gelu_tanh(v):
    return 0.5 * v * (1.0 + jnp.tanh(0.7978845608028654
                                     * (v + 0.044715 * v * v * v)))


def _k2_body(tp_ref, tq_ref, ea_ref, wc_ref, b1_ref, u0_ref, u1_ref):
    p0, p1 = _unpack_bf16_pair(tp_ref[...])
    q0, q1 = _unpack_bf16_pair(tq_ref[...])
    r = jnp.dot(ea_ref[...], wc_ref[...], preferred_element_type=jnp.float32)
    b1 = b1_ref[...]
    u0_ref[...] = _gelu_tanh(p0 + q0 + r[:, :D] + b1[:, :D])
    u1_ref[...] = _gelu_tanh(p1 + q1 + r[:, D:] + b1[:, D:])


_k2 = pl.pallas_call(
    _k2_body,
    grid=(EP // EBLK,),
    in_specs=[
        pl.BlockSpec((EBLK, H // 2), lambda i: (i, 0)),
        pl.BlockSpec((EBLK, H // 2), lambda i: (i, 0)),
        pl.BlockSpec((EBLK, ED), lambda i: (i, 0)),
        pl.BlockSpec((ED, H), lambda i: (0, 0)),
        pl.BlockSpec((1, H), lambda i: (0, 0)),
    ],
    out_specs=[
        pl.BlockSpec((EBLK, D), lambda i: (i, 0)),
        pl.BlockSpec((EBLK, D), lambda i: (i, 0)),
    ],
    out_shape=[jax.ShapeDtypeStruct((EP, D), jnp.float32)] * 2,
)


def _s2_body(u0_hbm, u1_hbm, col_hbm, part0_hbm, part1_hbm, pcnt_hbm,
             col_v, ub, acc, sem0, sem1):
    cid = lax.axis_index("c")
    sid = lax.axis_index("s")
    wid = cid * NSUB + sid
    pltpu.sync_copy(col_hbm.at[pl.ds(wid * CHT, CHT)], col_v)

    zero16 = jnp.zeros((16,), jnp.float32)
    ones16 = jnp.ones((16,), jnp.float32)
    sems = (sem0, sem1)

    def _fill(b, val):
        def frow(r, c2):
            for k in range(D // 16):
                ub[b, r, pl.ds(k * 16, 16)] = val
            return c2

        lax.fori_loop(0, C, frow, 0)

    for h in range(3):
        _fill(0, zero16)
        for k in range(RPT // C):
            pltpu.sync_copy(ub.at[0], acc.at[pl.ds(sid * RPT + k * C, C)])
        if h == 2:
            _fill(0, ones16)
        plsc.subcore_barrier()
        u_hbm = (u0_hbm, u1_hbm, None)[h]

        if u_hbm is None:
            def chunk(j, carry):
                pltpu.sync_copy(ub.at[0], acc.at[col_v.at[j]], add=True)
                return carry

            lax.fori_loop(0, CHT, chunk, 0)
        else:
            pltpu.async_copy(
                u_hbm.at[pl.ds(wid * CHT * C, C)], ub.at[0], sem0)

            def chunk2(jj, carry):
                for b in (0, 1):
                    j = jj * 2 + b

                    @pl.when(j + 1 < CHT)
                    def _():
                        pltpu.async_copy(
                            u_hbm.at[pl.ds((wid * CHT + j + 1) * C, C)],
                            ub.at[1 - b], sems[1 - b])

                    pltpu.make_async_copy(
                        u_hbm.at[pl.ds((wid * CHT + j) * C, C)],
                        ub.at[b], sems[b]).wait()
                    pltpu.sync_copy(ub.at[b], acc.at[col_v.at[j]], add=True)
                return carry

            lax.fori_loop(0, CHT // 2, chunk2, 0)
        plsc.subcore_barrier()
        part = (part0_hbm, part1_hbm, pcnt_hbm)[h]
        pltpu.sync_copy(acc.at[pl.ds(sid * RPT, RPT)],
                        part.at[cid, pl.ds(sid * RPT, RPT)])


_s2 = pl.kernel(
    _s2_body,
    out_type=(
        jax.ShapeDtypeStruct((2, NP, D), jnp.float32),
        jax.ShapeDtypeStruct((2, NP, D), jnp.float32),
        jax.ShapeDtypeStruct((2, NP, D), jnp.float32),
    ),
    mesh=plsc.VectorSubcoreMesh(core_axis_name="c", subcore_axis_name="s"),
    scratch_types=[
        pltpu.VMEM((CHT, C), jnp.int32),
        pltpu.VMEM((2, C, D), jnp.float32),
        pltpu.VMEM_SHARED((NP, D), jnp.float32),
        pltpu.SemaphoreType.DMA,
        pltpu.SemaphoreType.DMA,
    ],
)


def _k3_body(x_ref, p0_ref, p1_ref, pc_ref, w2_ref, b2_ref,
             ua_ref, uba_ref, ub2w_ref, ubb_ref, g_ref, be_ref, o_ref):
    s0 = p0_ref[0] + p0_ref[1]
    s1 = p1_ref[0] + p1_ref[1]
    cnt = pc_ref[0, :, 0:1] + pc_ref[1, :, 0:1]
    s = jnp.concatenate([s0, s1], axis=1)
    aggr = (jnp.dot(s, w2_ref[...], preferred_element_type=jnp.float32)
            + cnt * b2_ref[...]) / jnp.maximum(cnt, 1.0)
    xb = x_ref[...]
    u_in = jnp.concatenate([xb, aggr], axis=1)
    h2 = _gelu_exact(jnp.dot(u_in, ua_ref[...],
                             preferred_element_type=jnp.float32) + uba_ref[...])
    out = jnp.dot(h2, ub2w_ref[...],
                  preferred_element_type=jnp.float32) + ubb_ref[...]
    z = xb + out
    mu = jnp.mean(z, axis=1, keepdims=True)
    dz = z - mu
    var = jnp.mean(dz * dz, axis=1, keepdims=True)
    o_ref[...] = dz * lax.rsqrt(var + 1e-5) * g_ref[...] + be_ref[...]


_k3 = pl.pallas_call(
    _k3_body,
    grid=(NN // NBLK,),
    in_specs=[
        pl.BlockSpec((NBLK, D), lambda i: (i, 0)),
        pl.BlockSpec((2, NBLK, D), lambda i: (0, i, 0)),
        pl.BlockSpec((2, NBLK, D), lambda i: (0, i, 0)),
        pl.BlockSpec((2, NBLK, D), lambda i: (0, i, 0)),
        pl.BlockSpec((H, D), lambda i: (0, 0)),
        pl.BlockSpec((1, D), lambda i: (0, 0)),
        pl.BlockSpec((H, D), lambda i: (0, 0)),
        pl.BlockSpec((1, D), lambda i: (0, 0)),
        pl.BlockSpec((D, D), lambda i: (0, 0)),
        pl.BlockSpec((1, D), lambda i: (0, 0)),
        pl.BlockSpec((1, D), lambda i: (0, 0)),
        pl.BlockSpec((1, D), lambda i: (0, 0)),
    ],
    out_specs=pl.BlockSpec((NBLK, D), lambda i: (i, 0)),
    out_shape=jax.ShapeDtypeStruct((NN, D), jnp.float32),
)


def kernel(x, edge_index, edge_attr, W1, b1, W2, b2, U1, ub1, U2, ub2,
           gamma, beta):
    f32 = jnp.float32
    row = edge_index[0]
    col = edge_index[1]
    rowp = jnp.concatenate(
        [row, jnp.zeros((EP - EE,), jnp.int32)]).reshape(EP // C, C)
    colp = jnp.concatenate(
        [col, jnp.full((EP - EE,), NN, jnp.int32)]).reshape(EP // C, C)
    eap = jnp.concatenate([edge_attr, jnp.zeros((EP - EE, ED), f32)], axis=0)
    xp = jnp.zeros((NP, D), f32).at[:NN].set(x)
    W1a = W1[:D]
    W1b = W1[D:2 * D]
    W1c = W1[2 * D:]
    P, Q = _k1(xp, W1a, W1b)
    Tp, Tq = _s1(P, Q, rowp, colp)
    Uh0, Uh1 = _k2(Tp, Tq, eap, W1c, b1.reshape(1, H))
    p0, p1, pc = _s2(Uh0, Uh1, colp)
    return _k3(x, p0, p1, pc, W2, b2.reshape(1, D), U1, ub1.reshape(1, D),
               U2, ub2.reshape(1, D), gamma.reshape(1, D), beta.reshape(1, D))


# speedup vs baseline: 1.0565x; 1.0326x over previous
import functools

import jax
import jax.numpy as jnp
from jax import lax
from jax.experimental import pallas as pl
from jax.experimental.pallas import tpu as pltpu
from jax.experimental.pallas import tpu_sc as plsc

NN = 10000
D = 128
H = 256
ED = 16
EE = 320000
C = 128
CHT = 80
EP = 32 * CHT * C
CH0 = 104
CH1 = 2 * CHT - CH0
CHMX = max(CH0, CH1)
NP = 10240
NSUB = 16
RPT = NP // NSUB
EBLK = 4096
NBLK = 1000

_SQRT_HALF = 0.7071067811865476


def _gelu_exact(v):
    z = v * _SQRT_HALF
    a = jnp.abs(z)
    t = 1.0 / (1.0 + 0.3275911 * a)
    poly = t * (0.254829592 + t * (-0.284496736 + t * (
        1.421413741 + t * (-1.453152027 + t * 1.061405429))))
    erf_z = jnp.sign(z) * (1.0 - poly * jnp.exp(-a * a))
    return v * 0.5 * (1.0 + erf_z)


def _rtne_bf16_bits(x):
    u = lax.bitcast_convert_type(x, jnp.int32)
    return u + 0x7FFF + (lax.shift_right_logical(u, 16) & 1)

def _pack_bf16_pair(lo, hi):
    lo_b = lax.shift_right_logical(_rtne_bf16_bits(lo), 16)
    hi_b = _rtne_bf16_bits(hi) & jnp.int32(-65536)
    return lo_b | hi_b


def _unpack_bf16_pair(w):
    lo = lax.bitcast_convert_type(lax.shift_left(w, 16), jnp.float32)
    hi = lax.bitcast_convert_type(w & jnp.int32(-65536), jnp.float32)
    return lo, hi


def _k1_body(x_ref, wa_ref, wb_ref, p_ref, q_ref):
    xb = x_ref[...]
    p = jnp.dot(xb, wa_ref[...], preferred_element_type=jnp.float32)
    q = jnp.dot(xb, wb_ref[...], preferred_element_type=jnp.float32)
    p_ref[...] = _pack_bf16_pair(p[:, :D], p[:, D:])
    q_ref[...] = _pack_bf16_pair(q[:, :D], q[:, D:])


_k1 = pl.pallas_call(
    _k1_body,
    grid=(NP // 1024,),
    in_specs=[
        pl.BlockSpec((1024, D), lambda i: (i, 0)),
        pl.BlockSpec((D, H), lambda i: (0, 0)),
        pl.BlockSpec((D, H), lambda i: (0, 0)),
    ],
    out_specs=[
        pl.BlockSpec((1024, H // 2), lambda i: (i, 0)),
        pl.BlockSpec((1024, H // 2), lambda i: (i, 0)),
    ],
    out_shape=[jax.ShapeDtypeStruct((NP, H // 2), jnp.int32)] * 2,
)


def _s1_body(p_hbm, q_hbm, row_hbm, col_hbm, tp_hbm, tq_hbm,
             row_v, col_v, pg, qg, sem_p0, sem_q0, sem_p1, sem_q1, sem_o):
    cid = lax.axis_index("c")
    sid = lax.axis_index("s")
    cbase = jnp.where(cid == 0, sid * CH0, NSUB * CH0 + sid * CH1)
    nch = jnp.where(cid == 0, CH0, CH1)

    @pl.when(cid == 0)
    def _():
        pltpu.sync_copy(row_hbm.at[pl.ds(cbase, CH0)], row_v.at[pl.ds(0, CH0)])
        pltpu.sync_copy(col_hbm.at[pl.ds(cbase, CH0)], col_v.at[pl.ds(0, CH0)])

    if CH1 > 0:
        @pl.when(cid == 1)
        def _():
            pltpu.sync_copy(row_hbm.at[pl.ds(cbase, CH1)],
                            row_v.at[pl.ds(0, CH1)])
            pltpu.sync_copy(col_hbm.at[pl.ds(cbase, CH1)],
                            col_v.at[pl.ds(0, CH1)])

    sems = ((sem_p0, sem_q0), (sem_p1, sem_q1))

    @pl.when(nch > 0)
    def _():
        pltpu.async_copy(p_hbm.at[row_v.at[0]], pg.at[0], sem_p0)
        pltpu.async_copy(q_hbm.at[col_v.at[0]], qg.at[0], sem_q0)

    def chunk2(jj, carry):
        for b in (0, 1):
            j = jj * 2 + b
            nb = 1 - b

            @pl.when(j + 1 < nch)
            def _():
                pltpu.async_copy(
                    p_hbm.at[row_v.at[j + 1]], pg.at[nb], sems[nb][0])
                pltpu.async_copy(
                    q_hbm.at[col_v.at[j + 1]], qg.at[nb], sems[nb][1])

            pltpu.make_async_copy(
                p_hbm.at[row_v.at[j]], pg.at[b], sems[b][0]).wait()
            pltpu.make_async_copy(
                q_hbm.at[col_v.at[j]], qg.at[b], sems[b][1]).wait()
            co1 = pltpu.async_copy(
                pg.at[b], tp_hbm.at[pl.ds((cbase + j) * C, C)], sem_o)
            co2 = pltpu.async_copy(
                qg.at[b], tq_hbm.at[pl.ds((cbase + j) * C, C)], sem_o)
            co1.wait()
            co2.wait()
        return carry

    lax.fori_loop(0, nch // 2, chunk2, 0)


_s1 = pl.kernel(
    _s1_body,
    out_type=(
        jax.ShapeDtypeStruct((EP, H // 2), jnp.int32),
        jax.ShapeDtypeStruct((EP, H // 2), jnp.int32),
    ),
    mesh=plsc.VectorSubcoreMesh(core_axis_name="c", subcore_axis_name="s"),
    scratch_types=[
        pltpu.VMEM((CHMX, C), jnp.int32),
        pltpu.VMEM((CHMX, C), jnp.int32),
        pltpu.VMEM((2, C, H // 2), jnp.int32),
        pltpu.VMEM((2, C, H // 2), jnp.int32),
        pltpu.SemaphoreType.DMA,
        pltpu.SemaphoreType.DMA,
        pltpu.SemaphoreType.DMA,
        pltpu.SemaphoreType.DMA,
        pltpu.SemaphoreType.DMA,
    ],
)


def _gelu_tanh(v):
    return 0.5 * v * (1.0 + jnp.tanh(0.7978845608028654
                                     * (v + 0.044715 * v * v * v)))


def _k2_body(tp_ref, tq_ref, ea_ref, wc_ref, b1_ref, u0_ref, u1_ref):
    p0, p1 = _unpack_bf16_pair(tp_ref[...])
    q0, q1 = _unpack_bf16_pair(tq_ref[...])
    r = jnp.dot(ea_ref[...], wc_ref[...], preferred_element_type=jnp.float32)
    b1 = b1_ref[...]
    u0_ref[...] = _gelu_tanh(p0 + q0 + r[:, :D] + b1[:, :D])
    u1_ref[...] = _gelu_tanh(p1 + q1 + r[:, D:] + b1[:, D:])


_k2 = pl.pallas_call(
    _k2_body,
    grid=(EP // EBLK,),
    in_specs=[
        pl.BlockSpec((EBLK, H // 2), lambda i: (i, 0)),
        pl.BlockSpec((EBLK, H // 2), lambda i: (i, 0)),
        pl.BlockSpec((EBLK, ED), lambda i: (i, 0)),
        pl.BlockSpec((ED, H), lambda i: (0, 0)),
        pl.BlockSpec((1, H), lambda i: (0, 0)),
    ],
    out_specs=[
        pl.BlockSpec((EBLK, D), lambda i: (i, 0)),
        pl.BlockSpec((EBLK, D), lambda i: (i, 0)),
    ],
    out_shape=[jax.ShapeDtypeStruct((EP, D), jnp.float32)] * 2,
)


def _s2_body(u0_hbm, u1_hbm, col_hbm, part0_hbm, part1_hbm, pcnt_hbm,
             col_v, ub, acc, sem0, sem1):
    cid = lax.axis_index("c")
    sid = lax.axis_index("s")
    wid = cid * NSUB + sid
    pltpu.sync_copy(col_hbm.at[pl.ds(wid * CHT, CHT)], col_v)

    zero16 = jnp.zeros((16,), jnp.float32)
    ones16 = jnp.ones((16,), jnp.float32)
    sems = (sem0, sem1)

    def _fill(b, val):
        def frow(r, c2):
            for k in range(D // 16):
                ub[b, r, pl.ds(k * 16, 16)] = val
            return c2

        lax.fori_loop(0, C, frow, 0)

    for h in range(3):
        _fill(0, zero16)
        for k in range(RPT // C):
            pltpu.sync_copy(ub.at[0], acc.at[pl.ds(sid * RPT + k * C, C)])
        if h == 2:
            _fill(0, ones16)
        plsc.subcore_barrier()
        u_hbm = (u0_hbm, u1_hbm, None)[h]

        if u_hbm is None:
            def chunk(j, carry):
                pltpu.sync_copy(ub.at[0], acc.at[col_v.at[j]], add=True)
                return carry

            lax.fori_loop(0, CHT, chunk, 0)
        else:
            pltpu.async_copy(
                u_hbm.at[pl.ds(wid * CHT * C, C)], ub.at[0], sem0)

            def chunk2(jj, carry):
                for b in (0, 1):
                    j = jj * 2 + b

                    @pl.when(j + 1 < CHT)
                    def _():
                        pltpu.async_copy(
                            u_hbm.at[pl.ds((wid * CHT + j + 1) * C, C)],
                            ub.at[1 - b], sems[1 - b])

                    pltpu.make_async_copy(
                        u_hbm.at[pl.ds((wid * CHT + j) * C, C)],
                        ub.at[b], sems[b]).wait()
                    pltpu.sync_copy(ub.at[b], acc.at[col_v.at[j]], add=True)
                return carry

            lax.fori_loop(0, CHT // 2, chunk2, 0)
        plsc.subcore_barrier()
        part = (part0_hbm, part1_hbm, pcnt_hbm)[h]
        pltpu.sync_copy(acc.at[pl.ds(sid * RPT, RPT)],
                        part.at[cid, pl.ds(sid * RPT, RPT)])


_s2 = pl.kernel(
    _s2_body,
    out_type=(
        jax.ShapeDtypeStruct((2, NP, D), jnp.float32),
        jax.ShapeDtypeStruct((2, NP, D), jnp.float32),
        jax.ShapeDtypeStruct((2, NP, D), jnp.float32),
    ),
    mesh=plsc.VectorSubcoreMesh(core_axis_name="c", subcore_axis_name="s"),
    scratch_types=[
        pltpu.VMEM((CHT, C), jnp.int32),
        pltpu.VMEM((2, C, D), jnp.float32),
        pltpu.VMEM_SHARED((NP, D), jnp.float32),
        pltpu.SemaphoreType.DMA,
        pltpu.SemaphoreType.DMA,
    ],
)


def _k3_body(x_ref, p0_ref, p1_ref, pc_ref, w2_ref, b2_ref,
             ua_ref, uba_ref, ub2w_ref, ubb_ref, g_ref, be_ref, o_ref):
    s0 = p0_ref[0] + p0_ref[1]
    s1 = p1_ref[0] + p1_ref[1]
    cnt = pc_ref[0, :, 0:1] + pc_ref[1, :, 0:1]
    s = jnp.concatenate([s0, s1], axis=1)
    aggr = (jnp.dot(s, w2_ref[...], preferred_element_type=jnp.float32)
            + cnt * b2_ref[...]) / jnp.maximum(cnt, 1.0)
    xb = x_ref[...]
    u_in = jnp.concatenate([xb, aggr], axis=1)
    h2 = _gelu_exact(jnp.dot(u_in, ua_ref[...],
                             preferred_element_type=jnp.float32) + uba_ref[...])
    out = jnp.dot(h2, ub2w_ref[...],
                  preferred_element_type=jnp.float32) + ubb_ref[...]
    z = xb + out
    mu = jnp.mean(z, axis=1, keepdims=True)
    dz = z - mu
    var = jnp.mean(dz * dz, axis=1, keepdims=True)
    o_ref[...] = dz * lax.rsqrt(var + 1e-5) * g_ref[...] + be_ref[...]


_k3 = pl.pallas_call(
    _k3_body,
    grid=(NN // NBLK,),
    in_specs=[
        pl.BlockSpec((NBLK, D), lambda i: (i, 0)),
        pl.BlockSpec((2, NBLK, D), lambda i: (0, i, 0)),
        pl.BlockSpec((2, NBLK, D), lambda i: (0, i, 0)),
        pl.BlockSpec((2, NBLK, D), lambda i: (0, i, 0)),
        pl.BlockSpec((H, D), lambda i: (0, 0)),
        pl.BlockSpec((1, D), lambda i: (0, 0)),
        pl.BlockSpec((H, D), lambda i: (0, 0)),
        pl.BlockSpec((1, D), lambda i: (0, 0)),
        pl.BlockSpec((D, D), lambda i: (0, 0)),
        pl.BlockSpec((1, D), lambda i: (0, 0)),
        pl.BlockSpec((1, D), lambda i: (0, 0)),
        pl.BlockSpec((1, D), lambda i: (0, 0)),
    ],
    out_specs=pl.BlockSpec((NBLK, D), lambda i: (i, 0)),
    out_shape=jax.ShapeDtypeStruct((NN, D), jnp.float32),
)


def kernel(x, edge_index, edge_attr, W1, b1, W2, b2, U1, ub1, U2, ub2,
           gamma, beta):
    f32 = jnp.float32
    row = edge_index[0]
    col = edge_index[1]
    rowp = jnp.concatenate(
        [row, jnp.zeros((EP - EE,), jnp.int32)]).reshape(EP // C, C)
    colp = jnp.concatenate(
        [col, jnp.full((EP - EE,), NN, jnp.int32)]).reshape(EP // C, C)
    eap = jnp.concatenate([edge_attr, jnp.zeros((EP - EE, ED), f32)], axis=0)
    xp = jnp.zeros((NP, D), f32).at[:NN].set(x)
    W1a = W1[:D]
    W1b = W1[D:2 * D]
    W1c = W1[2 * D:]
    P, Q = _k1(xp, W1a, W1b)
    Tp, Tq = _s1(P, Q, rowp, colp)
    Uh0, Uh1 = _k2(Tp, Tq, eap, W1c, b1.reshape(1, H))
    p0, p1, pc = _s2(Uh0, Uh1, colp)
    return _k3(x, p0, p1, pc, W2, b2.reshape(1, D), U1, ub1.reshape(1, D),
               U2, ub2.reshape(1, D), gamma.reshape(1, D), beta.reshape(1, D))


# speedup vs baseline: 1.0596x; 1.0029x over previous
import functools

import jax
import jax.numpy as jnp
from jax import lax
from jax.experimental import pallas as pl
from jax.experimental.pallas import tpu as pltpu
from jax.experimental.pallas import tpu_sc as plsc

NN = 10000
D = 128
H = 256
ED = 16
EE = 320000
C = 128
CHT = 80
EP = 32 * CHT * C
CH0 = 104
CH1 = 2 * CHT - CH0
CHMX = max(CH0, CH1)
NP = 10240
NSUB = 16
RPT = NP // NSUB
EBLK = 8192
NBLK = 1000

_SQRT_HALF = 0.7071067811865476


def _gelu_exact(v):
    z = v * _SQRT_HALF
    a = jnp.abs(z)
    t = 1.0 / (1.0 + 0.3275911 * a)
    poly = t * (0.254829592 + t * (-0.284496736 + t * (
        1.421413741 + t * (-1.453152027 + t * 1.061405429))))
    erf_z = jnp.sign(z) * (1.0 - poly * jnp.exp(-a * a))
    return v * 0.5 * (1.0 + erf_z)


def _rtne_bf16_bits(x):
    u = lax.bitcast_convert_type(x, jnp.int32)
    return u + 0x7FFF + (lax.shift_right_logical(u, 16) & 1)

def _pack_bf16_pair(lo, hi):
    lo_b = lax.shift_right_logical(_rtne_bf16_bits(lo), 16)
    hi_b = _rtne_bf16_bits(hi) & jnp.int32(-65536)
    return lo_b | hi_b


def _unpack_bf16_pair(w):
    lo = lax.bitcast_convert_type(lax.shift_left(w, 16), jnp.float32)
    hi = lax.bitcast_convert_type(w & jnp.int32(-65536), jnp.float32)
    return lo, hi


def _k1_body(x_ref, wa_ref, wb_ref, p_ref, q_ref):
    xb = x_ref[...]
    p = jnp.dot(xb, wa_ref[...], preferred_element_type=jnp.float32)
    q = jnp.dot(xb, wb_ref[...], preferred_element_type=jnp.float32)
    p_ref[...] = _pack_bf16_pair(p[:, :D], p[:, D:])
    q_ref[...] = _pack_bf16_pair(q[:, :D], q[:, D:])


_k1 = pl.pallas_call(
    _k1_body,
    grid=(NP // 1024,),
    in_specs=[
        pl.BlockSpec((1024, D), lambda i: (i, 0)),
        pl.BlockSpec((D, H), lambda i: (0, 0)),
        pl.BlockSpec((D, H), lambda i: (0, 0)),
    ],
    out_specs=[
        pl.BlockSpec((1024, H // 2), lambda i: (i, 0)),
        pl.BlockSpec((1024, H // 2), lambda i: (i, 0)),
    ],
    out_shape=[jax.ShapeDtypeStruct((NP, H // 2), jnp.int32)] * 2,
)


def _s1_body(p_hbm, q_hbm, row_hbm, col_hbm, tp_hbm, tq_hbm,
             row_v, col_v, pg, qg, sem_p0, sem_q0, sem_p1, sem_q1, sem_o):
    cid = lax.axis_index("c")
    sid = lax.axis_index("s")
    cbase = jnp.where(cid == 0, sid * CH0, NSUB * CH0 + sid * CH1)
    nch = jnp.where(cid == 0, CH0, CH1)

    @pl.when(cid == 0)
    def _():
        pltpu.sync_copy(row_hbm.at[pl.ds(cbase, CH0)], row_v.at[pl.ds(0, CH0)])
        pltpu.sync_copy(col_hbm.at[pl.ds(cbase, CH0)], col_v.at[pl.ds(0, CH0)])

    if CH1 > 0:
        @pl.when(cid == 1)
        def _():
            pltpu.sync_copy(row_hbm.at[pl.ds(cbase, CH1)],
                            row_v.at[pl.ds(0, CH1)])
            pltpu.sync_copy(col_hbm.at[pl.ds(cbase, CH1)],
                            col_v.at[pl.ds(0, CH1)])

    sems = ((sem_p0, sem_q0), (sem_p1, sem_q1))

    @pl.when(nch > 0)
    def _():
        pltpu.async_copy(p_hbm.at[row_v.at[0]], pg.at[0], sem_p0)
        pltpu.async_copy(q_hbm.at[col_v.at[0]], qg.at[0], sem_q0)

    def chunk2(jj, carry):
        for b in (0, 1):
            j = jj * 2 + b
            nb = 1 - b

            @pl.when(j + 1 < nch)
            def _():
                pltpu.async_copy(
                    p_hbm.at[row_v.at[j + 1]], pg.at[nb], sems[nb][0])
                pltpu.async_copy(
                    q_hbm.at[col_v.at[j + 1]], qg.at[nb], sems[nb][1])

            pltpu.make_async_copy(
                p_hbm.at[row_v.at[j]], pg.at[b], sems[b][0]).wait()
            pltpu.make_async_copy(
                q_hbm.at[col_v.at[j]], qg.at[b], sems[b][1]).wait()
            co1 = pltpu.async_copy(
                pg.at[b], tp_hbm.at[pl.ds((cbase + j) * C, C)], sem_o)
            co2 = pltpu.async_copy(
                qg.at[b], tq_hbm.at[pl.ds((cbase + j) * C, C)], sem_o)
            co1.wait()
            co2.wait()
        return carry

    lax.fori_loop(0, nch // 2, chunk2, 0)


_s1 = pl.kernel(
    _s1_body,
    out_type=(
        jax.ShapeDtypeStruct((EP, H // 2), jnp.int32),
        jax.ShapeDtypeStruct((EP, H // 2), jnp.int32),
    ),
    mesh=plsc.VectorSubcoreMesh(core_axis_name="c", subcore_axis_name="s"),
    scratch_types=[
        pltpu.VMEM((CHMX, C), jnp.int32),
        pltpu.VMEM((CHMX, C), jnp.int32),
        pltpu.VMEM((2, C, H // 2), jnp.int32),
        pltpu.VMEM((2, C, H // 2), jnp.int32),
        pltpu.SemaphoreType.DMA,
        pltpu.SemaphoreType.DMA,
        pltpu.SemaphoreType.DMA,
        pltpu.SemaphoreType.DMA,
        pltpu.SemaphoreType.DMA,
    ],
)


def _gelu_tanh(v):
    return 0.5 * v * (1.0 + jnp.tanh(0.7978845608028654
                                     * (v + 0.044715 * v * v * v)))


def _k2_body(tp_ref, tq_ref, ea_ref, wc_ref, b1_ref, u0_ref, u1_ref):
    p0, p1 = _unpack_bf16_pair(tp_ref[...])
    q0, q1 = _unpack_bf16_pair(tq_ref[...])
    r = jnp.dot(ea_ref[...], wc_ref[...], preferred_element_type=jnp.float32)
    b1 = b1_ref[...]
    u0_ref[...] = _gelu_tanh(p0 + q0 + r[:, :D] + b1[:, :D])
    u1_ref[...] = _gelu_tanh(p1 + q1 + r[:, D:] + b1[:, D:])


_k2 = pl.pallas_call(
    _k2_body,
    grid=(EP // EBLK,),
    in_specs=[
        pl.BlockSpec((EBLK, H // 2), lambda i: (i, 0)),
        pl.BlockSpec((EBLK, H // 2), lambda i: (i, 0)),
        pl.BlockSpec((EBLK, ED), lambda i: (i, 0)),
        pl.BlockSpec((ED, H), lambda i: (0, 0)),
        pl.BlockSpec((1, H), lambda i: (0, 0)),
    ],
    out_specs=[
        pl.BlockSpec((EBLK, D), lambda i: (i, 0)),
        pl.BlockSpec((EBLK, D), lambda i: (i, 0)),
    ],
    out_shape=[jax.ShapeDtypeStruct((EP, D), jnp.float32)] * 2,
)


def _s2_body(u0_hbm, u1_hbm, col_hbm, part0_hbm, part1_hbm, pcnt_hbm,
             col_v, ub, acc, sem0, sem1):
    cid = lax.axis_index("c")
    sid = lax.axis_index("s")
    wid = cid * NSUB + sid
    pltpu.sync_copy(col_hbm.at[pl.ds(wid * CHT, CHT)], col_v)

    zero16 = jnp.zeros((16,), jnp.float32)
    ones16 = jnp.ones((16,), jnp.float32)
    sems = (sem0, sem1)

    def _fill(b, val):
        def frow(r, c2):
            for k in range(D // 16):
                ub[b, r, pl.ds(k * 16, 16)] = val
            return c2

        lax.fori_loop(0, C, frow, 0)

    for h in range(3):
        _fill(0, zero16)
        for k in range(RPT // C):
            pltpu.sync_copy(ub.at[0], acc.at[pl.ds(sid * RPT + k * C, C)])
        if h == 2:
            _fill(0, ones16)
        plsc.subcore_barrier()
        u_hbm = (u0_hbm, u1_hbm, None)[h]

        if u_hbm is None:
            def chunk(j, carry):
                pltpu.sync_copy(ub.at[0], acc.at[col_v.at[j]], add=True)
                return carry

            lax.fori_loop(0, CHT, chunk, 0)
        else:
            pltpu.async_copy(
                u_hbm.at[pl.ds(wid * CHT * C, C)], ub.at[0], sem0)

            def chunk2(jj, carry):
                for b in (0, 1):
                    j = jj * 2 + b

                    @pl.when(j + 1 < CHT)
                    def _():
                        pltpu.async_copy(
                            u_hbm.at[pl.ds((wid * CHT + j + 1) * C, C)],
                            ub.at[1 - b], sems[1 - b])

                    pltpu.make_async_copy(
                        u_hbm.at[pl.ds((wid * CHT + j) * C, C)],
                        ub.at[b], sems[b]).wait()
                    pltpu.sync_copy(ub.at[b], acc.at[col_v.at[j]], add=True)
                return carry

            lax.fori_loop(0, CHT // 2, chunk2, 0)
        plsc.subcore_barrier()
        part = (part0_hbm, part1_hbm, pcnt_hbm)[h]
        pltpu.sync_copy(acc.at[pl.ds(sid * RPT, RPT)],
                        part.at[cid, pl.ds(sid * RPT, RPT)])


_s2 = pl.kernel(
    _s2_body,
    out_type=(
        jax.ShapeDtypeStruct((2, NP, D), jnp.float32),
        jax.ShapeDtypeStruct((2, NP, D), jnp.float32),
        jax.ShapeDtypeStruct((2, NP, D), jnp.float32),
    ),
    mesh=plsc.VectorSubcoreMesh(core_axis_name="c", subcore_axis_name="s"),
    scratch_types=[
        pltpu.VMEM((CHT, C), jnp.int32),
        pltpu.VMEM((2, C, D), jnp.float32),
        pltpu.VMEM_SHARED((NP, D), jnp.float32),
        pltpu.SemaphoreType.DMA,
        pltpu.SemaphoreType.DMA,
    ],
)


def _k3_body(x_ref, p0_ref, p1_ref, pc_ref, w2_ref, b2_ref,
             ua_ref, uba_ref, ub2w_ref, ubb_ref, g_ref, be_ref, o_ref):
    s0 = p0_ref[0] + p0_ref[1]
    s1 = p1_ref[0] + p1_ref[1]
    cnt = pc_ref[0, :, 0:1] + pc_ref[1, :, 0:1]
    s = jnp.concatenate([s0, s1], axis=1)
    aggr = (jnp.dot(s, w2_ref[...], preferred_element_type=jnp.float32)
            + cnt * b2_ref[...]) / jnp.maximum(cnt, 1.0)
    xb = x_ref[...]
    u_in = jnp.concatenate([xb, aggr], axis=1)
    h2 = _gelu_exact(jnp.dot(u_in, ua_ref[...],
                             preferred_element_type=jnp.float32) + uba_ref[...])
    out = jnp.dot(h2, ub2w_ref[...],
                  preferred_element_type=jnp.float32) + ubb_ref[...]
    z = xb + out
    mu = jnp.mean(z, axis=1, keepdims=True)
    dz = z - mu
    var = jnp.mean(dz * dz, axis=1, keepdims=True)
    o_ref[...] = dz * lax.rsqrt(var + 1e-5) * g_ref[...] + be_ref[...]


_k3 = pl.pallas_call(
    _k3_body,
    grid=(NN // NBLK,),
    in_specs=[
        pl.BlockSpec((NBLK, D), lambda i: (i, 0)),
        pl.BlockSpec((2, NBLK, D), lambda i: (0, i, 0)),
        pl.BlockSpec((2, NBLK, D), lambda i: (0, i, 0)),
        pl.BlockSpec((2, NBLK, D), lambda i: (0, i, 0)),
        pl.BlockSpec((H, D), lambda i: (0, 0)),
        pl.BlockSpec((1, D), lambda i: (0, 0)),
        pl.BlockSpec((H, D), lambda i: (0, 0)),
        pl.BlockSpec((1, D), lambda i: (0, 0)),
        pl.BlockSpec((D, D), lambda i: (0, 0)),
        pl.BlockSpec((1, D), lambda i: (0, 0)),
        pl.BlockSpec((1, D), lambda i: (0, 0)),
        pl.BlockSpec((1, D), lambda i: (0, 0)),
    ],
    out_specs=pl.BlockSpec((NBLK, D), lambda i: (i, 0)),
    out_shape=jax.ShapeDtypeStruct((NN, D), jnp.float32),
)


def kernel(x, edge_index, edge_attr, W1, b1, W2, b2, U1, ub1, U2, ub2,
           gamma, beta):
    f32 = jnp.float32
    row = edge_index[0]
    col = edge_index[1]
    rowp = jnp.concatenate(
        [row, jnp.zeros((EP - EE,), jnp.int32)]).reshape(EP // C, C)
    colp = jnp.concatenate(
        [col, jnp.full((EP - EE,), NN, jnp.int32)]).reshape(EP // C, C)
    eap = jnp.concatenate([edge_attr, jnp.zeros((EP - EE, ED), f32)], axis=0)
    xp = jnp.zeros((NP, D), f32).at[:NN].set(x)
    W1a = W1[:D]
    W1b = W1[D:2 * D]
    W1c = W1[2 * D:]
    P, Q = _k1(xp, W1a, W1b)
    Tp, Tq = _s1(P, Q, rowp, colp)
    Uh0, Uh1 = _k2(Tp, Tq, eap, W1c, b1.reshape(1, H))
    p0, p1, pc = _s2(Uh0, Uh1, colp)
    return _k3(x, p0, p1, pc, W2, b2.reshape(1, D), U1, ub1.reshape(1, D),
               U2, ub2.reshape(1, D), gamma.reshape(1, D), beta.reshape(1, D))
